# baseline (speedup 1.0000x reference)
import functools

import numpy as np
import jax
import jax.numpy as jnp
from jax import lax
from jax.experimental import pallas as pl
from jax.experimental.pallas import tpu as pltpu
from jax.experimental.pallas import tpu_sc as plsc

_N = 10000
_NPAD = 10240
_F = 128
_K = 10
_E = 320000
_NW = 32
_EPAD = 160 * 128 * 16
_SINKPAD = 10232
_NBT = 160
_BND = _NPAD // 2
_ACCR = _BND + 8
_NE = _EPAD // 128

_BR = 1024
_GRID = _NPAD // _BR

_j = np.arange(_K + 1, dtype=np.float64)
_xj = np.cos((_K - _j + 0.5) * np.pi / (_K + 1))
_T = np.cos(_j[:, None] * np.arccos(np.clip(_xj[None, :], -1.0, 1.0)))
_CT = np.zeros((128, 128), dtype=np.float32)
_CT[: _K + 1, : _K + 1] = (2.0 / (_K + 1)) * _T.T


def _zero_buf(buf):
    zv = jnp.zeros((16,), jnp.float32)

    def zrow(i, carry):
        for k in range(8):
            buf[i, pl.ds(16 * k, 16)] = zv
        return carry

    lax.fori_loop(0, 128, zrow, 0)


def _sc_prop_body(g_hbm, edges_hbm, out_hbm,
                  idx_v, rows0, rows1, acc, gsm, ssm, ssc):
    c = lax.axis_index("c")
    s = lax.axis_index("s")

    _zero_buf(rows0)

    def zacc(t, carry):
        pltpu.sync_copy(rows0.at[pl.ds(0, 64)],
                        acc.at[pl.ds(s * (_BND // 16) + 64 * t, 64)])
        return carry

    lax.fori_loop(0, _BND // 16 // 64, zacc, 0)

    def stage(t, carry):
        base = jnp.where(t < _NBT // 32, s * _NBT + 32 * t,
                         _NE + s * _NBT + 32 * (t - _NBT // 32))
        pltpu.sync_copy(edges_hbm.at[pl.ds(base, 32)],
                        idx_v.at[pl.ds(32 * t, 32)])
        return carry

    lax.fori_loop(0, 2 * (_NBT // 32), stage, 0)

    lo = c * _BND

    def prep(r, carry):
        for k in range(8):
            d16 = idx_v[_NBT + r, pl.ds(16 * k, 16)] - lo
            inw = (d16 >= 0) & (d16 < _BND)
            idx_v[_NBT + r, pl.ds(16 * k, 16)] = jnp.where(inw, d16, _BND)
        return carry

    lax.fori_loop(0, _NBT, prep, 0)
    plsc.subcore_barrier()

    pltpu.make_async_copy(g_hbm.at[idx_v.at[0]], rows0, gsm).start()

    def pair(jj, carry):
        j0 = 2 * jj
        j1 = j0 + 1
        pltpu.make_async_copy(g_hbm.at[idx_v.at[j1]], rows1, ssm).start()
        pltpu.make_async_copy(g_hbm.at[idx_v.at[j0]], rows0, gsm).wait()
        pltpu.async_copy(rows0, acc.at[idx_v.at[_NBT + j0]], ssc, add=True)
        pltpu.make_async_copy(g_hbm.at[idx_v.at[j1]], rows1, ssm).wait()
        pltpu.async_copy(rows1, acc.at[idx_v.at[_NBT + j1]], ssc, add=True)
        pltpu.make_async_copy(rows0, acc.at[idx_v.at[_NBT + j0]], ssc).wait()

        @pl.when(jj < _NBT // 2 - 1)
        def _():
            pltpu.make_async_copy(g_hbm.at[idx_v.at[j0 + 2]], rows0,
                                  gsm).start()

        pltpu.make_async_copy(rows1, acc.at[idx_v.at[_NBT + j1]], ssc).wait()
        return carry

    lax.fori_loop(0, _NBT // 2, pair, 0)
    plsc.subcore_barrier()

    pltpu.sync_copy(acc.at[pl.ds(s * (_BND // 16), _BND // 16)],
                    out_hbm.at[pl.ds(c * _BND + s * (_BND // 16),
                                     _BND // 16)])


_sc_prop = functools.partial(
    pl.kernel,
    out_type=jax.ShapeDtypeStruct((_NPAD, _F), jnp.float32),
    mesh=plsc.VectorSubcoreMesh(core_axis_name="c", subcore_axis_name="s"),
    scratch_types=[
        pltpu.VMEM((2 * _NBT, 128), jnp.int32),
        pltpu.VMEM((128, _F), jnp.float32),
        pltpu.VMEM((128, _F), jnp.float32),
        pltpu.VMEM_SHARED((_ACCR, _F), jnp.float32),
        pltpu.SemaphoreType.DMA,
        pltpu.SemaphoreType.DMA,
        pltpu.SemaphoreType.DMA,
    ],
)(_sc_prop_body)


def _coe_body(g_ref, ct_ref, o_ref):
    g = jnp.maximum(g_ref[...], 0.0)
    o_ref[...] = jnp.dot(g, ct_ref[...], preferred_element_type=jnp.float32)


def _h_body(x_ref, w_ref, b_ref, o_ref):
    o_ref[...] = lax.dot_general(
        x_ref[...], w_ref[...], (((1,), (1,)), ((), ())),
        preferred_element_type=jnp.float32) + b_ref[...]


def _prep_body(p_ref, h_ref, dinv_o, g_o):
    d = p_ref[...]
    dinv = jnp.where(d > 0.0, lax.rsqrt(jnp.abs(d) + 1e-30), 0.0)
    dinv_o[...] = dinv
    g_o[...] = dinv * h_ref[...]


def _merge1_body(p_ref, h_ref, dinv_ref, coe_ref,
                 tx_o, g_o, outl_o, outh_o):
    dinv = dinv_ref[...]
    tx1 = -dinv * p_ref[...]
    tx_o[...] = tx1
    g_o[...] = dinv * tx1
    h = h_ref[...]
    outl_o[...] = 0.5 * coe_ref[0:1, 0:1] * h + coe_ref[0:1, 1:2] * tx1
    outh_o[...] = 0.5 * coe_ref[1:2, 0:1] * h + coe_ref[1:2, 1:2] * tx1


def _make_merge_body(i):
    def body(p_ref, txm2_ref, dinv_ref, outl_ref, outh_ref, coe_ref,
             tx_o, g_o, outl_o, outh_o):
        dinv = dinv_ref[...]
        tx = -2.0 * dinv * p_ref[...] - txm2_ref[...]
        tx_o[...] = tx
        g_o[...] = dinv * tx
        outl_o[...] = outl_ref[...] + coe_ref[0:1, i:i + 1] * tx
        outh_o[...] = outh_ref[...] + coe_ref[1:2, i:i + 1] * tx
    return body


def _stats_body(ol_ref, oh_ref, st_o):
    i = pl.program_id(0)

    @pl.when(i == 0)
    def _():
        st_o[...] = jnp.zeros((8, 128), jnp.float32)

    rows = _BR * i + lax.broadcasted_iota(jnp.int32, (_BR, _F), 0)
    msk = rows < _N
    a = jnp.where(msk, ol_ref[...], 0.0)
    b = jnp.where(msk, oh_ref[...], 0.0)
    st_o[0:1, :] += jnp.sum(a, axis=0, keepdims=True)
    st_o[1:2, :] += jnp.sum(a * a, axis=0, keepdims=True)
    st_o[2:3, :] += jnp.sum(b, axis=0, keepdims=True)
    st_o[3:4, :] += jnp.sum(b * b, axis=0, keepdims=True)

    @pl.when(i == _GRID - 1)
    def _():
        st = st_o[...]
        n = jnp.float32(_N)
        mean_l = st[0:1, :] / n
        var_l = st[1:2, :] / n - mean_l * mean_l
        mean_h = st[2:3, :] / n
        var_h = st[3:4, :] / n - mean_h * mean_h
        rstd_l = lax.rsqrt(var_l + 1e-5)
        rstd_h = lax.rsqrt(var_h + 1e-5)
        st_o[...] = jnp.concatenate(
            [mean_l, rstd_l, mean_h, rstd_h, jnp.zeros((4, 128), jnp.float32)],
            axis=0)


def _make_final_body(ro):
    def body(o_ref, st_ref, gam_ref, bet_ref, wup_ref, bu_ref, pa_ref, z_o):
        mean = st_ref[ro:ro + 1, :]
        rstd = st_ref[ro + 1:ro + 2, :]
        xn = (o_ref[...] - mean) * rstd * gam_ref[...] + bet_ref[...]
        y = lax.dot_general(
            xn, wup_ref[...], (((1,), (1,)), ((), ())),
            preferred_element_type=jnp.float32) + bu_ref[...]
        z_o[...] = jnp.where(y > 0.0, y, pa_ref[...] * y)
    return body


def _blk(i):
    return pl.BlockSpec((_BR, _F), lambda i: (i, 0))


def _fix(shape):
    return pl.BlockSpec(shape, lambda i: (0,) * len(shape))


_f32 = jnp.float32


def kernel(x, edge_index, W_in, b_in, temp_low, temp_high,
           bn_gamma, bn_beta, W_up, b_up, prelu_a):
    src = edge_index[0].astype(jnp.int32)
    dst = edge_index[1].astype(jnp.int32)
    npad = _EPAD - _E
    zpad = jnp.zeros((npad,), jnp.int32)
    spad = jnp.full((npad,), _SINKPAD, jnp.int32)
    edges_p = jnp.concatenate([src, zpad, dst, spad]).reshape(2 * _NE, 128)
    edges_d = jnp.concatenate([dst, zpad, src, spad]).reshape(2 * _NE, 128)

    xp = jnp.concatenate([x, jnp.zeros((_NPAD - _N, _F), _f32)], axis=0)
    nf = jax.ShapeDtypeStruct((_NPAD, _F), _f32)

    ones_g = jnp.ones((_NPAD, _F), _f32)
    pdeg = _sc_prop(ones_g, edges_d)

    h = pl.pallas_call(
        _h_body,
        grid=(_GRID,),
        in_specs=[_blk(0), _fix((128, 128)), _fix((1, 128))],
        out_specs=_blk(0),
        out_shape=nf,
    )(xp, W_in, b_in[None, :])

    gamma_pad = jnp.zeros((8, 128), _f32)
    gamma_pad = gamma_pad.at[0, : _K + 1].set(temp_low)
    gamma_pad = gamma_pad.at[1, : _K + 1].set(temp_high)
    coe = pl.pallas_call(
        _coe_body,
        in_specs=[_fix((8, 128)), _fix((128, 128))],
        out_specs=_fix((8, 128)),
        out_shape=jax.ShapeDtypeStruct((8, 128), _f32),
        grid=(1,),
    )(gamma_pad, jnp.asarray(_CT))

    dinv, g = pl.pallas_call(
        _prep_body,
        grid=(_GRID,),
        in_specs=[_blk(0)] * 2,
        out_specs=[_blk(0)] * 2,
        out_shape=[nf, nf],
    )(pdeg, h)

    p = _sc_prop(g, edges_p)
    tx1, g, out_l, out_h = pl.pallas_call(
        _merge1_body,
        grid=(_GRID,),
        in_specs=[_blk(0)] * 3 + [_fix((8, 128))],
        out_specs=[_blk(0)] * 4,
        out_shape=[nf, nf, nf, nf],
    )(p, h, dinv, coe)

    txm2 = h
    for i in range(2, _K + 1):
        p = _sc_prop(g, edges_p)
        tx, g, out_l, out_h = pl.pallas_call(
            _make_merge_body(i),
            grid=(_GRID,),
            in_specs=[_blk(0)] * 5 + [_fix((8, 128))],
            out_specs=[_blk(0)] * 4,
            out_shape=[nf, nf, nf, nf],
        )(p, txm2, dinv, out_l, out_h, coe)
        txm2 = tx1
        tx1 = tx

    stats = pl.pallas_call(
        _stats_body,
        grid=(_GRID,),
        in_specs=[_blk(0)] * 2,
        out_specs=_fix((8, 128)),
        out_shape=jax.ShapeDtypeStruct((8, 128), _f32),
    )(out_l, out_h)

    gam = bn_gamma[None, :]
    bet = bn_beta[None, :]
    bu = b_up[None, :]
    pa = jnp.broadcast_to(prelu_a, (1, 128)).astype(_f32)
    outs = []
    for ro, o in ((0, out_l), (2, out_h)):
        z = pl.pallas_call(
            _make_final_body(ro),
            grid=(_GRID,),
            in_specs=[_blk(0), _fix((8, 128)), _fix((1, 128)),
                      _fix((1, 128)), _fix((128, 128)), _fix((1, 128)),
                      _fix((1, 128))],
            out_specs=_blk(0),
            out_shape=nf,
        )(o, stats, gam, bet, W_up, bu, pa)
        outs.append(z[:_N])
    return (outs[0], outs[1])


# speedup vs baseline: 1.0676x; 1.0676x over previous
import functools

import numpy as np
import jax
import jax.numpy as jnp
from jax import lax
from jax.experimental import pallas as pl
from jax.experimental.pallas import tpu as pltpu
from jax.experimental.pallas import tpu_sc as plsc

_N = 10000
_NPAD = 10240
_F = 128
_K = 10
_E = 320000
_NW = 32
_EPAD = 160 * 128 * 16
_SINKPAD = 10232
_NBT = 160
_BND = _NPAD // 2
_ACCR = _BND + 8
_NE = _EPAD // 128

_BR = 1024
_GRID = _NPAD // _BR

_j = np.arange(_K + 1, dtype=np.float64)
_xj = np.cos((_K - _j + 0.5) * np.pi / (_K + 1))
_T = np.cos(_j[:, None] * np.arccos(np.clip(_xj[None, :], -1.0, 1.0)))
_CT = np.zeros((128, 128), dtype=np.float32)
_CT[: _K + 1, : _K + 1] = (2.0 / (_K + 1)) * _T.T


def _zero_buf(buf):
    zv = jnp.zeros((16,), jnp.float32)

    def zrow(i, carry):
        for k in range(8):
            buf[i, pl.ds(16 * k, 16)] = zv
        return carry

    lax.fori_loop(0, 128, zrow, 0)


def _sc_prop_body(g_hbm, edges_hbm, out_hbm,
                  idx_v, rows0, rows1, acc, gsm, ssm, ssc):
    c = lax.axis_index("c")
    s = lax.axis_index("s")

    _zero_buf(rows0)

    def zacc(t, carry):
        pltpu.sync_copy(rows0.at[pl.ds(0, 64)],
                        acc.at[pl.ds(s * (_BND // 16) + 64 * t, 64)])
        return carry

    lax.fori_loop(0, _BND // 16 // 64, zacc, 0)

    def stage(t, carry):
        base = jnp.where(t < _NBT // 32, s * _NBT + 32 * t,
                         _NE + s * _NBT + 32 * (t - _NBT // 32))
        pltpu.sync_copy(edges_hbm.at[pl.ds(base, 32)],
                        idx_v.at[pl.ds(32 * t, 32)])
        return carry

    lax.fori_loop(0, 2 * (_NBT // 32), stage, 0)

    lo = c * _BND

    def prep(r, carry):
        for k in range(8):
            d16 = idx_v[_NBT + r, pl.ds(16 * k, 16)] - lo
            inw = (d16 >= 0) & (d16 < _BND)
            idx_v[_NBT + r, pl.ds(16 * k, 16)] = jnp.where(inw, d16, _BND)
        return carry

    lax.fori_loop(0, _NBT, prep, 0)
    plsc.subcore_barrier()

    pltpu.make_async_copy(g_hbm.at[idx_v.at[0]], rows0, gsm).start()

    def pair(jj, carry):
        j0 = 2 * jj
        j1 = j0 + 1
        pltpu.make_async_copy(g_hbm.at[idx_v.at[j1]], rows1, ssm).start()
        pltpu.make_async_copy(g_hbm.at[idx_v.at[j0]], rows0, gsm).wait()
        pltpu.async_copy(rows0, acc.at[idx_v.at[_NBT + j0]], ssc, add=True)
        pltpu.make_async_copy(g_hbm.at[idx_v.at[j1]], rows1, ssm).wait()
        pltpu.async_copy(rows1, acc.at[idx_v.at[_NBT + j1]], ssc, add=True)
        pltpu.make_async_copy(rows0, acc.at[idx_v.at[_NBT + j0]], ssc).wait()

        @pl.when(jj < _NBT // 2 - 1)
        def _():
            pltpu.make_async_copy(g_hbm.at[idx_v.at[j0 + 2]], rows0,
                                  gsm).start()

        pltpu.make_async_copy(rows1, acc.at[idx_v.at[_NBT + j1]], ssc).wait()
        return carry

    lax.fori_loop(0, _NBT // 2, pair, 0)
    plsc.subcore_barrier()

    pltpu.sync_copy(acc.at[pl.ds(s * (_BND // 16), _BND // 16)],
                    out_hbm.at[pl.ds(c * _BND + s * (_BND // 16),
                                     _BND // 16)])


def _sc_deg_body(edges_hbm, out_hbm, idx_v, rows0, acc, ssc):
    c = lax.axis_index("c")
    s = lax.axis_index("s")

    _zero_buf(rows0)

    def zacc(t, carry):
        pltpu.sync_copy(rows0.at[pl.ds(0, 64)],
                        acc.at[pl.ds(s * (_BND // 16) + 64 * t, 64)])
        return carry

    lax.fori_loop(0, _BND // 16 // 64, zacc, 0)

    def stage(t, carry):
        base = jnp.where(t < _NBT // 32, s * _NBT + 32 * t,
                         _NE + s * _NBT + 32 * (t - _NBT // 32))
        pltpu.sync_copy(edges_hbm.at[pl.ds(base, 32)],
                        idx_v.at[pl.ds(32 * t, 32)])
        return carry

    lax.fori_loop(0, 2 * (_NBT // 32), stage, 0)

    lo = c * _BND

    def prep(r, carry):
        for k in range(8):
            d16 = idx_v[_NBT + r, pl.ds(16 * k, 16)] - lo
            inw = (d16 >= 0) & (d16 < _BND)
            idx_v[_NBT + r, pl.ds(16 * k, 16)] = jnp.where(inw, d16, _BND)
        return carry

    lax.fori_loop(0, _NBT, prep, 0)

    ones16 = jnp.ones((16,), jnp.float32)

    def orow(i, carry):
        for k in range(8):
            rows0[i, pl.ds(16 * k, 16)] = ones16
        return carry

    lax.fori_loop(0, 128, orow, 0)
    plsc.subcore_barrier()

    def step(j, carry):
        pltpu.sync_copy(rows0, acc.at[idx_v.at[_NBT + j]], add=True)
        return carry

    lax.fori_loop(0, _NBT, step, 0)
    plsc.subcore_barrier()
    pltpu.sync_copy(acc.at[pl.ds(s * (_BND // 16), _BND // 16)],
                    out_hbm.at[pl.ds(c * _BND + s * (_BND // 16),
                                     _BND // 16)])


_sc_deg = functools.partial(
    pl.kernel,
    out_type=jax.ShapeDtypeStruct((_NPAD, _F), jnp.float32),
    mesh=plsc.VectorSubcoreMesh(core_axis_name="c", subcore_axis_name="s"),
    scratch_types=[
        pltpu.VMEM((2 * _NBT, 128), jnp.int32),
        pltpu.VMEM((128, _F), jnp.float32),
        pltpu.VMEM_SHARED((_ACCR, _F), jnp.float32),
        pltpu.SemaphoreType.DMA,
    ],
)(_sc_deg_body)


_sc_prop = functools.partial(
    pl.kernel,
    out_type=jax.ShapeDtypeStruct((_NPAD, _F), jnp.float32),
    mesh=plsc.VectorSubcoreMesh(core_axis_name="c", subcore_axis_name="s"),
    scratch_types=[
        pltpu.VMEM((2 * _NBT, 128), jnp.int32),
        pltpu.VMEM((128, _F), jnp.float32),
        pltpu.VMEM((128, _F), jnp.float32),
        pltpu.VMEM_SHARED((_ACCR, _F), jnp.float32),
        pltpu.SemaphoreType.DMA,
        pltpu.SemaphoreType.DMA,
        pltpu.SemaphoreType.DMA,
    ],
)(_sc_prop_body)


def _coe_body(g_ref, ct_ref, o_ref):
    g = jnp.maximum(g_ref[...], 0.0)
    o_ref[...] = jnp.dot(g, ct_ref[...], preferred_element_type=jnp.float32)


def _h_body(x_ref, w_ref, b_ref, o_ref):
    o_ref[...] = lax.dot_general(
        x_ref[...], w_ref[...], (((1,), (1,)), ((), ())),
        preferred_element_type=jnp.float32) + b_ref[...]


def _prep_body(p_ref, h_ref, dinv_o, g_o):
    d = p_ref[...]
    dinv = jnp.where(d > 0.0, lax.rsqrt(jnp.abs(d) + 1e-30), 0.0)
    dinv_o[...] = dinv
    g_o[...] = dinv * h_ref[...]


def _merge1_body(p_ref, h_ref, dinv_ref, coe_ref,
                 tx_o, g_o, outl_o, outh_o):
    dinv = dinv_ref[...]
    tx1 = -dinv * p_ref[...]
    tx_o[...] = tx1
    g_o[...] = dinv * tx1
    h = h_ref[...]
    outl_o[...] = 0.5 * coe_ref[0:1, 0:1] * h + coe_ref[0:1, 1:2] * tx1
    outh_o[...] = 0.5 * coe_ref[1:2, 0:1] * h + coe_ref[1:2, 1:2] * tx1


def _make_merge_body(i):
    def body(p_ref, txm2_ref, dinv_ref, outl_ref, outh_ref, coe_ref,
             tx_o, g_o, outl_o, outh_o):
        dinv = dinv_ref[...]
        tx = -2.0 * dinv * p_ref[...] - txm2_ref[...]
        tx_o[...] = tx
        g_o[...] = dinv * tx
        outl_o[...] = outl_ref[...] + coe_ref[0:1, i:i + 1] * tx
        outh_o[...] = outh_ref[...] + coe_ref[1:2, i:i + 1] * tx
    return body


def _stats_body(ol_ref, oh_ref, st_o):
    i = pl.program_id(0)

    @pl.when(i == 0)
    def _():
        st_o[...] = jnp.zeros((8, 128), jnp.float32)

    rows = _BR * i + lax.broadcasted_iota(jnp.int32, (_BR, _F), 0)
    msk = rows < _N
    a = jnp.where(msk, ol_ref[...], 0.0)
    b = jnp.where(msk, oh_ref[...], 0.0)
    st_o[0:1, :] += jnp.sum(a, axis=0, keepdims=True)
    st_o[1:2, :] += jnp.sum(a * a, axis=0, keepdims=True)
    st_o[2:3, :] += jnp.sum(b, axis=0, keepdims=True)
    st_o[3:4, :] += jnp.sum(b * b, axis=0, keepdims=True)

    @pl.when(i == _GRID - 1)
    def _():
        st = st_o[...]
        n = jnp.float32(_N)
        mean_l = st[0:1, :] / n
        var_l = st[1:2, :] / n - mean_l * mean_l
        mean_h = st[2:3, :] / n
        var_h = st[3:4, :] / n - mean_h * mean_h
        rstd_l = lax.rsqrt(var_l + 1e-5)
        rstd_h = lax.rsqrt(var_h + 1e-5)
        st_o[...] = jnp.concatenate(
            [mean_l, rstd_l, mean_h, rstd_h, jnp.zeros((4, 128), jnp.float32)],
            axis=0)


def _make_final_body(ro):
    def body(o_ref, st_ref, gam_ref, bet_ref, wup_ref, bu_ref, pa_ref, z_o):
        mean = st_ref[ro:ro + 1, :]
        rstd = st_ref[ro + 1:ro + 2, :]
        xn = (o_ref[...] - mean) * rstd * gam_ref[...] + bet_ref[...]
        y = lax.dot_general(
            xn, wup_ref[...], (((1,), (1,)), ((), ())),
            preferred_element_type=jnp.float32) + bu_ref[...]
        z_o[...] = jnp.where(y > 0.0, y, pa_ref[...] * y)
    return body


def _blk(i):
    return pl.BlockSpec((_BR, _F), lambda i: (i, 0))


def _fix(shape):
    return pl.BlockSpec(shape, lambda i: (0,) * len(shape))


_f32 = jnp.float32


def kernel(x, edge_index, W_in, b_in, temp_low, temp_high,
           bn_gamma, bn_beta, W_up, b_up, prelu_a):
    src = edge_index[0].astype(jnp.int32)
    dst = edge_index[1].astype(jnp.int32)
    npad = _EPAD - _E
    zpad = jnp.zeros((npad,), jnp.int32)
    spad = jnp.full((npad,), _SINKPAD, jnp.int32)
    edges_p = jnp.concatenate([src, zpad, dst, spad]).reshape(2 * _NE, 128)
    edges_d = jnp.concatenate([dst, zpad, src, spad]).reshape(2 * _NE, 128)

    xp = jnp.concatenate([x, jnp.zeros((_NPAD - _N, _F), _f32)], axis=0)
    nf = jax.ShapeDtypeStruct((_NPAD, _F), _f32)

    pdeg = _sc_deg(edges_d)

    h = pl.pallas_call(
        _h_body,
        grid=(_GRID,),
        in_specs=[_blk(0), _fix((128, 128)), _fix((1, 128))],
        out_specs=_blk(0),
        out_shape=nf,
    )(xp, W_in, b_in[None, :])

    gamma_pad = jnp.zeros((8, 128), _f32)
    gamma_pad = gamma_pad.at[0, : _K + 1].set(temp_low)
    gamma_pad = gamma_pad.at[1, : _K + 1].set(temp_high)
    coe = pl.pallas_call(
        _coe_body,
        in_specs=[_fix((8, 128)), _fix((128, 128))],
        out_specs=_fix((8, 128)),
        out_shape=jax.ShapeDtypeStruct((8, 128), _f32),
        grid=(1,),
    )(gamma_pad, jnp.asarray(_CT))

    dinv, g = pl.pallas_call(
        _prep_body,
        grid=(_GRID,),
        in_specs=[_blk(0)] * 2,
        out_specs=[_blk(0)] * 2,
        out_shape=[nf, nf],
    )(pdeg, h)

    p = _sc_prop(g, edges_p)
    tx1, g, out_l, out_h = pl.pallas_call(
        _merge1_body,
        grid=(_GRID,),
        in_specs=[_blk(0)] * 3 + [_fix((8, 128))],
        out_specs=[_blk(0)] * 4,
        out_shape=[nf, nf, nf, nf],
    )(p, h, dinv, coe)

    txm2 = h
    for i in range(2, _K + 1):
        p = _sc_prop(g, edges_p)
        tx, g, out_l, out_h = pl.pallas_call(
            _make_merge_body(i),
            grid=(_GRID,),
            in_specs=[_blk(0)] * 5 + [_fix((8, 128))],
            out_specs=[_blk(0)] * 4,
            out_shape=[nf, nf, nf, nf],
        )(p, txm2, dinv, out_l, out_h, coe)
        txm2 = tx1
        tx1 = tx

    stats = pl.pallas_call(
        _stats_body,
        grid=(_GRID,),
        in_specs=[_blk(0)] * 2,
        out_specs=_fix((8, 128)),
        out_shape=jax.ShapeDtypeStruct((8, 128), _f32),
    )(out_l, out_h)

    gam = bn_gamma[None, :]
    bet = bn_beta[None, :]
    bu = b_up[None, :]
    pa = jnp.broadcast_to(prelu_a, (1, 128)).astype(_f32)
    outs = []
    for ro, o in ((0, out_l), (2, out_h)):
        z = pl.pallas_call(
            _make_final_body(ro),
            grid=(_GRID,),
            in_specs=[_blk(0), _fix((8, 128)), _fix((1, 128)),
                      _fix((1, 128)), _fix((128, 128)), _fix((1, 128)),
                      _fix((1, 128))],
            out_specs=_blk(0),
            out_shape=nf,
        )(o, stats, gam, bet, W_up, bu, pa)
        outs.append(z[:_N])
    return (outs[0], outs[1])
